# Initial kernel scaffold; baseline (speedup 1.0000x reference)
import jax
import jax.numpy as jnp
from jax.experimental import pallas as pl


def kernel(x, edge_index, batch, params):
    raise NotImplementedError("write your pallas kernel here")


# speedup vs baseline: 1.8101x; 1.8101x over previous
import functools

import jax
import jax.numpy as jnp
from jax import lax
from jax.experimental import pallas as pl
from jax.experimental.pallas import tpu as pltpu
from jax.experimental.pallas import tpu_sc as plsc

_G = 50
_NC, _NS = 2, 16
_NW = _NC * _NS
_NPAD = 50176
_NPW = _NPAD // _NW
_EPAD = 800256
_EPW = _EPAD // _NW
_ECHUNK = 8336
_PADNODE = 50047
_PADG = _G

_mesh = plsc.VectorSubcoreMesh(core_axis_name="c", subcore_axis_name="s")


def _wid():
    return lax.axis_index("s") * _NC + lax.axis_index("c")


def _zero(ref, n):
    def body(i, c):
        ref[pl.ds(i * 16, 16)] = jnp.zeros((16,), ref.dtype)
        return c
    lax.fori_loop(0, n // 16, body, 0)


@functools.partial(
    pl.kernel,
    out_type=jax.ShapeDtypeStruct((_NW, _NPAD), jnp.float32),
    mesh=_mesh,
    compiler_params=pltpu.CompilerParams(needs_layout_passes=False),
    scratch_types=[pltpu.VMEM((_NPAD,), jnp.float32),
                   pltpu.VMEM((_EPW,), jnp.int32)],
)
def _sc_deg(row_hbm, out_hbm, acc_v, idx_v):
    w = _wid()
    pltpu.sync_copy(row_hbm.at[pl.ds(w * _EPW, _EPW)], idx_v)
    _zero(acc_v, _NPAD)
    ones = jnp.ones((16,), jnp.float32)

    def body(i, c):
        r = idx_v[pl.ds(i * 16, 16)]
        plsc.addupdate_scatter(acc_v, [r], ones)
        return c
    lax.fori_loop(0, _EPW // 16, body, 0, unroll=8)
    pltpu.sync_copy(acc_v, out_hbm.at[w])


@functools.partial(
    pl.kernel,
    out_type=jax.ShapeDtypeStruct((_NW, 64), jnp.float32),
    mesh=_mesh,
    compiler_params=pltpu.CompilerParams(needs_layout_passes=False),
    scratch_types=[pltpu.VMEM((_NPAD,), jnp.float32),
                   pltpu.VMEM((_NPAD,), jnp.int32),
                   pltpu.VMEM((_ECHUNK,), jnp.int32),
                   pltpu.VMEM((_ECHUNK,), jnp.int32),
                   pltpu.VMEM((64,), jnp.float32)],
)
def _sc_edge(vals_hbm, batch_hbm, row_hbm, col_hbm, out_hbm,
             vals_v, batch_v, row_v, col_v, acc_v):
    w = _wid()
    pltpu.sync_copy(vals_hbm, vals_v)
    pltpu.sync_copy(batch_hbm, batch_v)
    _zero(acc_v, 64)
    base = w * _EPW

    def chunk(k, c):
        pltpu.sync_copy(row_hbm.at[pl.ds(base + k * _ECHUNK, _ECHUNK)], row_v)
        pltpu.sync_copy(col_hbm.at[pl.ds(base + k * _ECHUNK, _ECHUNK)], col_v)

        def body(i, c2):
            sl = pl.ds(i * 16, 16)
            r = row_v[sl]
            cc = col_v[sl]
            pr = plsc.load_gather(vals_v, [r])
            pc = plsc.load_gather(vals_v, [cc])
            b = plsc.load_gather(batch_v, [r])
            contrib = jnp.where(r != cc, pr * pc, jnp.zeros((16,), jnp.float32))
            plsc.addupdate_scatter(acc_v, [b], contrib)
            return c2
        lax.fori_loop(0, _ECHUNK // 16, body, c, unroll=8)
        return c
    lax.fori_loop(0, _EPW // _ECHUNK, chunk, 0)
    pltpu.sync_copy(acc_v, out_hbm.at[w])


@functools.partial(
    pl.kernel,
    out_type=jax.ShapeDtypeStruct((_NW, 320), jnp.float32),
    mesh=_mesh,
    compiler_params=pltpu.CompilerParams(needs_layout_passes=False),
    scratch_types=[pltpu.VMEM((_NPW,), jnp.int32),
                   pltpu.VMEM((_NPW,), jnp.float32),
                   pltpu.VMEM((_NPW,), jnp.float32),
                   pltpu.VMEM((_NPW,), jnp.float32),
                   pltpu.VMEM((320,), jnp.float32)],
)
def _sc_nodes(batch_hbm, p_hbm, x2_hbm, deg_hbm, out_hbm,
              b_v, p_v, x2_v, deg_v, acc_v):
    w = _wid()
    base = w * _NPW
    pltpu.sync_copy(batch_hbm.at[pl.ds(base, _NPW)], b_v)
    pltpu.sync_copy(p_hbm.at[pl.ds(base, _NPW)], p_v)
    pltpu.sync_copy(x2_hbm.at[pl.ds(base, _NPW)], x2_v)
    pltpu.sync_copy(deg_hbm.at[pl.ds(base, _NPW)], deg_v)
    _zero(acc_v, 320)

    def body(i, c):
        sl = pl.ds(i * 16, 16)
        b = b_v[sl]
        p = p_v[sl]
        x = x2_v[sl]
        d = deg_v[sl]
        plsc.addupdate_scatter(acc_v, [b], p)
        plsc.addupdate_scatter(acc_v, [b + 64], p * p)
        plsc.addupdate_scatter(acc_v, [b + 128], x)
        plsc.addupdate_scatter(acc_v, [b + 192], d)
        plsc.addupdate_scatter(acc_v, [b + 256], d * x)
        return c
    lax.fori_loop(0, _NPW // 16, body, 0, unroll=4)
    pltpu.sync_copy(acc_v, out_hbm.at[w])


def _tc1_body(hf_ref, b_ref, u_ref, degb_ref, p_ref, x2_ref, deg_ref):
    hf = hf_ref[...]
    b = b_ref[...]
    u = u_ref[...]
    deg_ref[...] = jnp.sum(degb_ref[...], axis=0)
    bmax_n = jnp.zeros_like(hf)
    bmin_n = jnp.zeros_like(hf)
    ninf = jnp.float32(-jnp.inf)
    pinf = jnp.float32(jnp.inf)
    for g in range(_G):
        m = b == g
        mx = jnp.max(jnp.where(m, hf, ninf))
        mn = jnp.min(jnp.where(m, hf, pinf))
        bmax_n = jnp.where(m, mx, bmax_n)
        bmin_n = jnp.where(m, mn, bmin_n)
    probs = (hf - bmin_n) / (bmax_n + 1e-6 - bmin_n)
    p_ref[...] = probs
    x2_ref[...] = (probs - u > 0).astype(jnp.float32)


def _tc1(hf2, batch2, u2, degb):
    return pl.pallas_call(
        _tc1_body,
        out_shape=(jax.ShapeDtypeStruct((392, 128), jnp.float32),
                   jax.ShapeDtypeStruct((392, 128), jnp.float32),
                   jax.ShapeDtypeStruct((392, 128), jnp.float32)),
    )(hf2, batch2, u2, degb)


def _tc2_body(nb_ref, eb_ref, sb_ref, out_ref):
    nb = jnp.sum(nb_ref[...], axis=0)
    esum = jnp.sum(eb_ref[...], axis=0, keepdims=True)
    ssum = jnp.sum(sb_ref[...], axis=0, keepdims=True)
    lane = jax.lax.broadcasted_iota(jnp.int32, (1, 64), 1)
    valid = lane < _G
    card_1 = nb[0:1, :]
    self_sums = nb[1:2, :]
    set_size = nb[2:3, :]
    totalvol = nb[3:4, :] + 1e-6
    vol_hard = nb[4:5, :] + 1e-6
    graph_sums = card_1
    pairwise_prodsums = graph_sums * graph_sums / 2.0
    ewg = esum / 2.0
    sw = ssum / 2.0 + 1e-6
    ecw = pairwise_prodsums - self_sums
    edist = ecw - ewg
    ceh = set_size * (set_size - 1.0) / 2.0 + 1e-6
    cdh = sw / ceh
    tvr = vol_hard / totalvol
    loss = 0.25 * edist * 0.5 - 0.5 * ewg

    def vmean(v):
        return jnp.sum(jnp.where(valid, v, 0.0)) / jnp.float32(_G)

    ewg_mean = vmean(ewg)
    edist_mean = vmean(edist)
    tvr_mean = vmean(tvr)
    loss_mean = vmean(loss)
    scal = jnp.where(lane == 0, ewg_mean, 0.0)
    scal = jnp.where(lane == 1, edist_mean, scal)
    scal = jnp.where(lane == 2, tvr_mean, scal)
    scal = jnp.where(lane == 3, loss_mean, scal)
    out_ref[...] = jnp.concatenate(
        [card_1, loss, set_size, cdh, scal,
         ewg, edist, tvr], axis=0)


def _tc2(node_bins, ewg_bins, sw_bins):
    return pl.pallas_call(
        _tc2_body,
        out_shape=jax.ShapeDtypeStruct((8, 64), jnp.float32),
    )(node_bins, ewg_bins, sw_bins)


def _bn(h, gamma, beta):
    mu = h.mean(axis=0)
    var = h.var(axis=0)
    return (h - mu) / jnp.sqrt(var + 1e-5) * gamma + beta


def _gin_conv(h, row, col, p, n):
    agg = jax.ops.segment_sum(h[row], col, num_segments=n)
    z = (1.0 + p["eps"]) * h + agg
    z = jax.nn.relu(z @ p["W1"] + p["b1"])
    z = jax.nn.relu(z @ p["W2"] + p["b2"])
    return _bn(z, p["gamma"], p["beta"])


def _get_mask(m, row, col, n):
    prop = jax.ops.segment_max(m[row], col, num_segments=n)
    prop = jnp.where(jnp.isfinite(prop), prop, 0.0)
    return (prop > 0).astype(jnp.float32)


def _gnorm(h, batch, num_graphs):
    cnt = jax.ops.segment_sum(jnp.ones((h.shape[0],), jnp.float32), batch, num_segments=num_graphs)
    inv = 1.0 / jnp.sqrt(jnp.maximum(cnt, 1.0))
    return h * inv[batch][:, None]


def kernel(x, edge_index, batch, params):
    key = jax.random.key(42)
    row, col = edge_index[0], edge_index[1]
    n = x.shape[0]
    xx = x[:, None]
    mask = _get_mask(xx, row, col, n)
    h = jax.nn.leaky_relu(_gin_conv(xx, row, col, params["conv1"], n))
    h = h * mask
    h = _gnorm(h, batch, _G)
    h = _bn(h, params["bn1"]["gamma"], params["bn1"]["beta"])
    for cp, bp in zip(params["convs"], params["bns"]):
        h = h + jax.nn.leaky_relu(_gin_conv(h, row, col, cp, n))
        mask = _get_mask(mask, row, col, n)
        h = h * mask
        h = _gnorm(h, batch, _G)
        h = _bn(h, bp["gamma"], bp["beta"])
    h = jax.nn.leaky_relu(h @ params["lin1"]["W"] + params["lin1"]["b"])
    h = h * mask
    h = jax.nn.leaky_relu(h @ params["lin2"]["W"] + params["lin2"]["b"])
    hf = h * mask

    epad = _EPAD - row.shape[0]
    rowp = jnp.concatenate([row, jnp.full((epad,), _PADNODE, jnp.int32)])
    colp = jnp.concatenate([col, jnp.full((epad,), _PADNODE, jnp.int32)])
    npad = _NPAD - n
    batchp = jnp.concatenate([batch, jnp.full((npad,), _PADG, jnp.int32)])
    hfp = jnp.concatenate([hf[:, 0], jnp.zeros((npad,), jnp.float32)])
    u = jax.random.uniform(key, (n, 1), jnp.float32)
    up = jnp.concatenate([u[:, 0], jnp.zeros((npad,), jnp.float32)])

    deg_bins = _sc_deg(rowp)
    probs2, x22, deg2 = _tc1(hfp.reshape(392, 128),
                             batchp.reshape(392, 128),
                             up.reshape(392, 128),
                             deg_bins.reshape(_NW, 392, 128))
    probs_f = probs2.reshape(_NPAD)
    x2_f = x22.reshape(_NPAD)
    deg_f = deg2.reshape(_NPAD)
    ewg_bins = _sc_edge(probs_f, batchp, rowp, colp)
    sw_bins = _sc_edge(x2_f, batchp, rowp, colp)
    node_bins = _sc_nodes(batchp, probs_f, x2_f, deg_f)
    out8 = _tc2(node_bins.reshape(_NW, 5, 64), ewg_bins, sw_bins)

    probs_out = probs_f[:n]
    card_1 = out8[0, :_G]
    loss = out8[1, :_G]
    set_size = out8[2, :_G]
    cdh = out8[3, :_G]
    ewg_mean = out8[4, 0]
    edist_mean = out8[4, 1]
    tvr_mean = out8[4, 2]
    loss_mean = out8[4, 3]
    return (probs_out, card_1, loss, set_size, ewg_mean, edist_mean,
            cdh, tvr_mean, loss_mean)


# speedup vs baseline: 3.2777x; 1.8108x over previous
import functools

import jax
import jax.numpy as jnp
from jax import lax
from jax.experimental import pallas as pl
from jax.experimental.pallas import tpu as pltpu
from jax.experimental.pallas import tpu_sc as plsc

_G = 50
_NC, _NS = 2, 16
_NW = _NC * _NS
_NPAD = 50176
_NPW = _NPAD // _NW
_EPAD = 800256
_EPW = _EPAD // _NW
_ECHUNK = 8336
_PADNODE = 50047
_PADG = _G

_mesh = plsc.VectorSubcoreMesh(core_axis_name="c", subcore_axis_name="s")


def _wid():
    return lax.axis_index("s") * _NC + lax.axis_index("c")


def _zero(ref, n):
    def body(i, c):
        ref[pl.ds(i * 16, 16)] = jnp.zeros((16,), ref.dtype)
        return c
    lax.fori_loop(0, n // 16, body, 0)


@functools.partial(
    pl.kernel,
    out_type=jax.ShapeDtypeStruct((_NW, _NPAD), jnp.float32),
    mesh=_mesh,
    compiler_params=pltpu.CompilerParams(needs_layout_passes=False),
    scratch_types=[pltpu.VMEM((_NPAD,), jnp.float32),
                   pltpu.VMEM((_EPW,), jnp.int32)],
)
def _sc_deg(row_hbm, out_hbm, acc_v, idx_v):
    w = _wid()
    pltpu.sync_copy(row_hbm.at[pl.ds(w * _EPW, _EPW)], idx_v)
    _zero(acc_v, _NPAD)
    ones = jnp.ones((16,), jnp.float32)

    def body(i, c):
        r = idx_v[pl.ds(i * 16, 16)]
        plsc.addupdate_scatter(acc_v, [r], ones)
        return c
    lax.fori_loop(0, _EPW // 16, body, 0, unroll=8)
    pltpu.sync_copy(acc_v, out_hbm.at[w])


@functools.partial(
    pl.kernel,
    out_type=jax.ShapeDtypeStruct((_NW, 64), jnp.float32),
    mesh=_mesh,
    compiler_params=pltpu.CompilerParams(needs_layout_passes=False),
    scratch_types=[pltpu.VMEM((_NPAD,), jnp.float32),
                   pltpu.VMEM((_NPAD,), jnp.int32),
                   pltpu.VMEM((_ECHUNK,), jnp.int32),
                   pltpu.VMEM((_ECHUNK,), jnp.int32),
                   pltpu.VMEM((64,), jnp.float32)],
)
def _sc_edge(vals_hbm, batch_hbm, row_hbm, col_hbm, out_hbm,
             vals_v, batch_v, row_v, col_v, acc_v):
    w = _wid()
    pltpu.sync_copy(vals_hbm, vals_v)
    pltpu.sync_copy(batch_hbm, batch_v)
    _zero(acc_v, 64)
    base = w * _EPW

    def chunk(k, c):
        pltpu.sync_copy(row_hbm.at[pl.ds(base + k * _ECHUNK, _ECHUNK)], row_v)
        pltpu.sync_copy(col_hbm.at[pl.ds(base + k * _ECHUNK, _ECHUNK)], col_v)

        def body(i, c2):
            sl = pl.ds(i * 16, 16)
            r = row_v[sl]
            cc = col_v[sl]
            pr = plsc.load_gather(vals_v, [r])
            pc = plsc.load_gather(vals_v, [cc])
            b = plsc.load_gather(batch_v, [r])
            contrib = jnp.where(r != cc, pr * pc, jnp.zeros((16,), jnp.float32))
            plsc.addupdate_scatter(acc_v, [b], contrib)
            return c2
        lax.fori_loop(0, _ECHUNK // 16, body, c, unroll=8)
        return c
    lax.fori_loop(0, _EPW // _ECHUNK, chunk, 0)
    pltpu.sync_copy(acc_v, out_hbm.at[w])


@functools.partial(
    pl.kernel,
    out_type=jax.ShapeDtypeStruct((_NW, 320), jnp.float32),
    mesh=_mesh,
    compiler_params=pltpu.CompilerParams(needs_layout_passes=False),
    scratch_types=[pltpu.VMEM((_NPW,), jnp.int32),
                   pltpu.VMEM((_NPW,), jnp.float32),
                   pltpu.VMEM((_NPW,), jnp.float32),
                   pltpu.VMEM((_NPW,), jnp.float32),
                   pltpu.VMEM((320,), jnp.float32)],
)
def _sc_nodes(batch_hbm, p_hbm, x2_hbm, deg_hbm, out_hbm,
              b_v, p_v, x2_v, deg_v, acc_v):
    w = _wid()
    base = w * _NPW
    pltpu.sync_copy(batch_hbm.at[pl.ds(base, _NPW)], b_v)
    pltpu.sync_copy(p_hbm.at[pl.ds(base, _NPW)], p_v)
    pltpu.sync_copy(x2_hbm.at[pl.ds(base, _NPW)], x2_v)
    pltpu.sync_copy(deg_hbm.at[pl.ds(base, _NPW)], deg_v)
    _zero(acc_v, 320)

    def body(i, c):
        sl = pl.ds(i * 16, 16)
        b = b_v[sl]
        p = p_v[sl]
        x = x2_v[sl]
        d = deg_v[sl]
        plsc.addupdate_scatter(acc_v, [b], p)
        plsc.addupdate_scatter(acc_v, [b + 64], p * p)
        plsc.addupdate_scatter(acc_v, [b + 128], x)
        plsc.addupdate_scatter(acc_v, [b + 192], d)
        plsc.addupdate_scatter(acc_v, [b + 256], d * x)
        return c
    lax.fori_loop(0, _NPW // 16, body, 0, unroll=4)
    pltpu.sync_copy(acc_v, out_hbm.at[w])


_MNPT = _NPAD // _NS

@functools.partial(
    pl.kernel,
    out_type=jax.ShapeDtypeStruct((4 * _NPAD,), jnp.float32),
    mesh=_mesh,
    compiler_params=pltpu.CompilerParams(needs_layout_passes=False),
    scratch_types=[pltpu.VMEM((_NPAD,), jnp.float32),
                   pltpu.VMEM((_ECHUNK,), jnp.int32),
                   pltpu.VMEM((_ECHUNK,), jnp.int32),
                   pltpu.VMEM((_MNPT,), jnp.float32)],
)
def _sc_masks(x_hbm, row_hbm, col_hbm, out_hbm, tab_v, row_v, col_v, acc_v):
    core = lax.axis_index("c")
    tid = lax.axis_index("s")
    lo = tid * _MNPT
    for k in range(4):
        @pl.when(core == 0)
        def _round():
            if k == 0:
                pltpu.sync_copy(x_hbm, tab_v)
            else:
                pltpu.sync_copy(out_hbm.at[pl.ds((k - 1) * _NPAD, _NPAD)], tab_v)
            _zero(acc_v, _MNPT)

            def chunk(j, c):
                pltpu.sync_copy(row_hbm.at[pl.ds(j * _ECHUNK, _ECHUNK)], row_v)
                pltpu.sync_copy(col_hbm.at[pl.ds(j * _ECHUNK, _ECHUNK)], col_v)

                def body(i, c2):
                    sl = pl.ds(i * 16, 16)
                    r = row_v[sl]
                    cc = col_v[sl]
                    v = plsc.load_gather(tab_v, [r])
                    ind = (v > 0).astype(jnp.float32)
                    idx = cc - lo
                    m = (cc >= lo) & (cc < lo + _MNPT)
                    plsc.addupdate_scatter(acc_v, [idx], ind, mask=m)
                    return c2
                lax.fori_loop(0, _ECHUNK // 16, body, c, unroll=8)
                return c
            lax.fori_loop(0, _EPAD // _ECHUNK, chunk, 0)
            pltpu.sync_copy(acc_v, out_hbm.at[pl.ds(k * _NPAD + lo, _MNPT)])
        plsc.subcore_barrier()


def _tc1_body(hf_ref, b_ref, u_ref, degb_ref, p_ref, x2_ref, deg_ref):
    hf = hf_ref[...]
    b = b_ref[...]
    u = u_ref[...]
    deg_ref[...] = jnp.sum(degb_ref[...], axis=0)
    bmax_n = jnp.zeros_like(hf)
    bmin_n = jnp.zeros_like(hf)
    ninf = jnp.float32(-jnp.inf)
    pinf = jnp.float32(jnp.inf)
    for g in range(_G):
        m = b == g
        mx = jnp.max(jnp.where(m, hf, ninf))
        mn = jnp.min(jnp.where(m, hf, pinf))
        bmax_n = jnp.where(m, mx, bmax_n)
        bmin_n = jnp.where(m, mn, bmin_n)
    probs = (hf - bmin_n) / (bmax_n + 1e-6 - bmin_n)
    p_ref[...] = probs
    x2_ref[...] = (probs - u > 0).astype(jnp.float32)


def _tc1(hf2, batch2, u2, degb):
    return pl.pallas_call(
        _tc1_body,
        out_shape=(jax.ShapeDtypeStruct((392, 128), jnp.float32),
                   jax.ShapeDtypeStruct((392, 128), jnp.float32),
                   jax.ShapeDtypeStruct((392, 128), jnp.float32)),
    )(hf2, batch2, u2, degb)


def _tc2_body(nb_ref, eb_ref, sb_ref, out_ref):
    nb = jnp.sum(nb_ref[...], axis=0)
    esum = jnp.sum(eb_ref[...], axis=0, keepdims=True)
    ssum = jnp.sum(sb_ref[...], axis=0, keepdims=True)
    lane = jax.lax.broadcasted_iota(jnp.int32, (1, 64), 1)
    valid = lane < _G
    card_1 = nb[0:1, :]
    self_sums = nb[1:2, :]
    set_size = nb[2:3, :]
    totalvol = nb[3:4, :] + 1e-6
    vol_hard = nb[4:5, :] + 1e-6
    graph_sums = card_1
    pairwise_prodsums = graph_sums * graph_sums / 2.0
    ewg = esum / 2.0
    sw = ssum / 2.0 + 1e-6
    ecw = pairwise_prodsums - self_sums
    edist = ecw - ewg
    ceh = set_size * (set_size - 1.0) / 2.0 + 1e-6
    cdh = sw / ceh
    tvr = vol_hard / totalvol
    loss = 0.25 * edist * 0.5 - 0.5 * ewg

    def vmean(v):
        return jnp.sum(jnp.where(valid, v, 0.0)) / jnp.float32(_G)

    ewg_mean = vmean(ewg)
    edist_mean = vmean(edist)
    tvr_mean = vmean(tvr)
    loss_mean = vmean(loss)
    scal = jnp.where(lane == 0, ewg_mean, 0.0)
    scal = jnp.where(lane == 1, edist_mean, scal)
    scal = jnp.where(lane == 2, tvr_mean, scal)
    scal = jnp.where(lane == 3, loss_mean, scal)
    out_ref[...] = jnp.concatenate(
        [card_1, loss, set_size, cdh, scal,
         ewg, edist, tvr], axis=0)


def _tc2(node_bins, ewg_bins, sw_bins):
    return pl.pallas_call(
        _tc2_body,
        out_shape=jax.ShapeDtypeStruct((8, 64), jnp.float32),
    )(node_bins, ewg_bins, sw_bins)


def _bn(h, gamma, beta):
    mu = h.mean(axis=0)
    var = h.var(axis=0)
    return (h - mu) / jnp.sqrt(var + 1e-5) * gamma + beta


def _gin_conv(h, row, col, p, n):
    agg = jax.ops.segment_sum(h[row], col, num_segments=n)
    z = (1.0 + p["eps"]) * h + agg
    z = jax.nn.relu(z @ p["W1"] + p["b1"])
    z = jax.nn.relu(z @ p["W2"] + p["b2"])
    return _bn(z, p["gamma"], p["beta"])


def _get_mask(m, row, col, n):
    prop = jax.ops.segment_max(m[row], col, num_segments=n)
    prop = jnp.where(jnp.isfinite(prop), prop, 0.0)
    return (prop > 0).astype(jnp.float32)


def _gnorm(h, batch, num_graphs):
    cnt = jax.ops.segment_sum(jnp.ones((h.shape[0],), jnp.float32), batch, num_segments=num_graphs)
    inv = 1.0 / jnp.sqrt(jnp.maximum(cnt, 1.0))
    return h * inv[batch][:, None]


def kernel(x, edge_index, batch, params):
    key = jax.random.key(42)
    row, col = edge_index[0], edge_index[1]
    n = x.shape[0]
    epad = _EPAD - row.shape[0]
    npad = _NPAD - n
    rowp = jnp.concatenate([row, jnp.full((epad,), _PADNODE, jnp.int32)])
    colp = jnp.concatenate([col, jnp.full((epad,), _PADNODE, jnp.int32)])
    xp = jnp.concatenate([x, jnp.zeros((npad,), jnp.float32)])
    masks_flat = _sc_masks(xp, rowp, colp)

    def mk(k):
        return (masks_flat[k * _NPAD:k * _NPAD + n] > 0).astype(jnp.float32)[:, None]

    xx = x[:, None]
    mask = mk(0)
    h = jax.nn.leaky_relu(_gin_conv(xx, row, col, params["conv1"], n))
    h = h * mask
    h = _gnorm(h, batch, _G)
    h = _bn(h, params["bn1"]["gamma"], params["bn1"]["beta"])
    for li, (cp, bp) in enumerate(zip(params["convs"], params["bns"])):
        h = h + jax.nn.leaky_relu(_gin_conv(h, row, col, cp, n))
        mask = mk(li + 1)
        h = h * mask
        h = _gnorm(h, batch, _G)
        h = _bn(h, bp["gamma"], bp["beta"])
    h = jax.nn.leaky_relu(h @ params["lin1"]["W"] + params["lin1"]["b"])
    h = h * mask
    h = jax.nn.leaky_relu(h @ params["lin2"]["W"] + params["lin2"]["b"])
    hf = h * mask

    batchp = jnp.concatenate([batch, jnp.full((npad,), _PADG, jnp.int32)])
    hfp = jnp.concatenate([hf[:, 0], jnp.zeros((npad,), jnp.float32)])
    u = jax.random.uniform(key, (n, 1), jnp.float32)
    up = jnp.concatenate([u[:, 0], jnp.zeros((npad,), jnp.float32)])

    deg_bins = _sc_deg(rowp)
    probs2, x22, deg2 = _tc1(hfp.reshape(392, 128),
                             batchp.reshape(392, 128),
                             up.reshape(392, 128),
                             deg_bins.reshape(_NW, 392, 128))
    probs_f = probs2.reshape(_NPAD)
    x2_f = x22.reshape(_NPAD)
    deg_f = deg2.reshape(_NPAD)
    ewg_bins = _sc_edge(probs_f, batchp, rowp, colp)
    sw_bins = _sc_edge(x2_f, batchp, rowp, colp)
    node_bins = _sc_nodes(batchp, probs_f, x2_f, deg_f)
    out8 = _tc2(node_bins.reshape(_NW, 5, 64), ewg_bins, sw_bins)

    probs_out = probs_f[:n]
    card_1 = out8[0, :_G]
    loss = out8[1, :_G]
    set_size = out8[2, :_G]
    cdh = out8[3, :_G]
    ewg_mean = out8[4, 0]
    edist_mean = out8[4, 1]
    tvr_mean = out8[4, 2]
    loss_mean = out8[4, 3]
    return (probs_out, card_1, loss, set_size, ewg_mean, edist_mean,
            cdh, tvr_mean, loss_mean)


# speedup vs baseline: 3.2836x; 1.0018x over previous
import functools

import jax
import jax.numpy as jnp
from jax import lax
from jax.experimental import pallas as pl
from jax.experimental.pallas import tpu as pltpu
from jax.experimental.pallas import tpu_sc as plsc

_G = 50
_NC, _NS = 2, 16
_NW = _NC * _NS
_NPAD = 50176
_NPW = _NPAD // _NW
_EPAD = 800256
_EPW = _EPAD // _NW
_ECHUNK = 8336
_PADNODE = 50047
_PADG = _G

_mesh = plsc.VectorSubcoreMesh(core_axis_name="c", subcore_axis_name="s")


def _wid():
    return lax.axis_index("s") * _NC + lax.axis_index("c")


def _zero(ref, n):
    def body(i, c):
        ref[pl.ds(i * 16, 16)] = jnp.zeros((16,), ref.dtype)
        return c
    lax.fori_loop(0, n // 16, body, 0)


@functools.partial(
    pl.kernel,
    out_type=jax.ShapeDtypeStruct((_NW, _NPAD), jnp.float32),
    mesh=_mesh,
    compiler_params=pltpu.CompilerParams(needs_layout_passes=False),
    scratch_types=[pltpu.VMEM((_NPAD,), jnp.float32),
                   pltpu.VMEM((_EPW,), jnp.int32)],
)
def _sc_deg(row_hbm, out_hbm, acc_v, idx_v):
    w = _wid()
    pltpu.sync_copy(row_hbm.at[pl.ds(w * _EPW, _EPW)], idx_v)
    _zero(acc_v, _NPAD)
    ones = jnp.ones((16,), jnp.float32)

    def body(i, c):
        r = idx_v[pl.ds(i * 16, 16)]
        plsc.addupdate_scatter(acc_v, [r], ones)
        return c
    lax.fori_loop(0, _EPW // 16, body, 0, unroll=8)
    pltpu.sync_copy(acc_v, out_hbm.at[w])


@functools.partial(
    pl.kernel,
    out_type=jax.ShapeDtypeStruct((_NW, 64), jnp.float32),
    mesh=_mesh,
    compiler_params=pltpu.CompilerParams(needs_layout_passes=False),
    scratch_types=[pltpu.VMEM((_NPAD,), jnp.float32),
                   pltpu.VMEM((_NPAD,), jnp.int32),
                   pltpu.VMEM((_ECHUNK,), jnp.int32),
                   pltpu.VMEM((_ECHUNK,), jnp.int32),
                   pltpu.VMEM((64,), jnp.float32)],
)
def _sc_edge(vals_hbm, batch_hbm, row_hbm, col_hbm, out_hbm,
             vals_v, batch_v, row_v, col_v, acc_v):
    w = _wid()
    pltpu.sync_copy(vals_hbm, vals_v)
    pltpu.sync_copy(batch_hbm, batch_v)
    _zero(acc_v, 64)
    base = w * _EPW

    def chunk(k, c):
        pltpu.sync_copy(row_hbm.at[pl.ds(base + k * _ECHUNK, _ECHUNK)], row_v)
        pltpu.sync_copy(col_hbm.at[pl.ds(base + k * _ECHUNK, _ECHUNK)], col_v)

        def body(i, c2):
            sl = pl.ds(i * 16, 16)
            r = row_v[sl]
            cc = col_v[sl]
            pr = plsc.load_gather(vals_v, [r])
            pc = plsc.load_gather(vals_v, [cc])
            b = plsc.load_gather(batch_v, [r])
            contrib = jnp.where(r != cc, pr * pc, jnp.zeros((16,), jnp.float32))
            plsc.addupdate_scatter(acc_v, [b], contrib)
            return c2
        lax.fori_loop(0, _ECHUNK // 16, body, c, unroll=8)
        return c
    lax.fori_loop(0, _EPW // _ECHUNK, chunk, 0)
    pltpu.sync_copy(acc_v, out_hbm.at[w])


@functools.partial(
    pl.kernel,
    out_type=jax.ShapeDtypeStruct((_NW, 320), jnp.float32),
    mesh=_mesh,
    compiler_params=pltpu.CompilerParams(needs_layout_passes=False),
    scratch_types=[pltpu.VMEM((_NPW,), jnp.int32),
                   pltpu.VMEM((_NPW,), jnp.float32),
                   pltpu.VMEM((_NPW,), jnp.float32),
                   pltpu.VMEM((_NPW,), jnp.float32),
                   pltpu.VMEM((320,), jnp.float32)],
)
def _sc_nodes(batch_hbm, p_hbm, x2_hbm, deg_hbm, out_hbm,
              b_v, p_v, x2_v, deg_v, acc_v):
    w = _wid()
    base = w * _NPW
    pltpu.sync_copy(batch_hbm.at[pl.ds(base, _NPW)], b_v)
    pltpu.sync_copy(p_hbm.at[pl.ds(base, _NPW)], p_v)
    pltpu.sync_copy(x2_hbm.at[pl.ds(base, _NPW)], x2_v)
    pltpu.sync_copy(deg_hbm.at[pl.ds(base, _NPW)], deg_v)
    _zero(acc_v, 320)

    def body(i, c):
        sl = pl.ds(i * 16, 16)
        b = b_v[sl]
        p = p_v[sl]
        x = x2_v[sl]
        d = deg_v[sl]
        plsc.addupdate_scatter(acc_v, [b], p)
        plsc.addupdate_scatter(acc_v, [b + 64], p * p)
        plsc.addupdate_scatter(acc_v, [b + 128], x)
        plsc.addupdate_scatter(acc_v, [b + 192], d)
        plsc.addupdate_scatter(acc_v, [b + 256], d * x)
        return c
    lax.fori_loop(0, _NPW // 16, body, 0, unroll=4)
    pltpu.sync_copy(acc_v, out_hbm.at[w])


_MNPT = _NPAD // _NS

@functools.partial(
    pl.kernel,
    out_type=jax.ShapeDtypeStruct((4 * _NPAD,), jnp.float32),
    mesh=_mesh,
    compiler_params=pltpu.CompilerParams(needs_layout_passes=False),
    scratch_types=[pltpu.VMEM((_NPAD,), jnp.float32),
                   pltpu.VMEM((_ECHUNK,), jnp.int32),
                   pltpu.VMEM((_ECHUNK,), jnp.int32),
                   pltpu.VMEM((_MNPT,), jnp.float32)],
)
def _sc_masks(x_hbm, row_hbm, col_hbm, out_hbm, tab_v, row_v, col_v, acc_v):
    core = lax.axis_index("c")
    tid = lax.axis_index("s")
    lo = tid * _MNPT
    for k in range(4):
        @pl.when(core == 0)
        def _round():
            if k == 0:
                pltpu.sync_copy(x_hbm, tab_v)
            else:
                pltpu.sync_copy(out_hbm.at[pl.ds((k - 1) * _NPAD, _NPAD)], tab_v)
            _zero(acc_v, _MNPT)

            def chunk(j, c):
                pltpu.sync_copy(row_hbm.at[pl.ds(j * _ECHUNK, _ECHUNK)], row_v)
                pltpu.sync_copy(col_hbm.at[pl.ds(j * _ECHUNK, _ECHUNK)], col_v)

                def body(i, c2):
                    sl = pl.ds(i * 16, 16)
                    r = row_v[sl]
                    cc = col_v[sl]
                    v = plsc.load_gather(tab_v, [r])
                    ind = (v > 0).astype(jnp.float32)
                    idx = cc - lo
                    m = (cc >= lo) & (cc < lo + _MNPT)
                    plsc.addupdate_scatter(acc_v, [idx], ind, mask=m)
                    return c2
                lax.fori_loop(0, _ECHUNK // 16, body, c, unroll=8)
                return c
            lax.fori_loop(0, _EPAD // _ECHUNK, chunk, 0)
            pltpu.sync_copy(acc_v, out_hbm.at[pl.ds(k * _NPAD + lo, _MNPT)])
        plsc.subcore_barrier()


@functools.partial(
    pl.kernel,
    out_type=jax.ShapeDtypeStruct((_NW, 64), jnp.float32),
    mesh=_mesh,
    compiler_params=pltpu.CompilerParams(needs_layout_passes=False),
    scratch_types=[pltpu.VMEM((_NPW,), jnp.int32),
                   pltpu.VMEM((64,), jnp.float32)],
)
def _sc_cnt(batch_hbm, out_hbm, b_v, acc_v):
    w = _wid()
    pltpu.sync_copy(batch_hbm.at[pl.ds(w * _NPW, _NPW)], b_v)
    _zero(acc_v, 64)
    ones = jnp.ones((16,), jnp.float32)

    def body(i, c):
        b = b_v[pl.ds(i * 16, 16)]
        plsc.addupdate_scatter(acc_v, [b], ones)
        return c
    lax.fori_loop(0, _NPW // 16, body, 0, unroll=8)
    pltpu.sync_copy(acc_v, out_hbm.at[w])


def _tcinv_body(b_ref, inv_ref, o_ref):
    b = b_ref[...]
    inv = inv_ref[...]
    out = jnp.zeros((392, 128), jnp.float32)
    for g in range(_G):
        out = jnp.where(b == g, inv[0, g], out)
    o_ref[...] = out


def _tcinv(batch2, inv64):
    return pl.pallas_call(
        _tcinv_body,
        out_shape=jax.ShapeDtypeStruct((392, 128), jnp.float32),
    )(batch2, inv64)


def _tc1_body(hf_ref, b_ref, u_ref, degb_ref, p_ref, x2_ref, deg_ref):
    hf = hf_ref[...]
    b = b_ref[...]
    u = u_ref[...]
    deg_ref[...] = jnp.sum(degb_ref[...], axis=0)
    bmax_n = jnp.zeros_like(hf)
    bmin_n = jnp.zeros_like(hf)
    ninf = jnp.float32(-jnp.inf)
    pinf = jnp.float32(jnp.inf)
    for g in range(_G):
        m = b == g
        mx = jnp.max(jnp.where(m, hf, ninf))
        mn = jnp.min(jnp.where(m, hf, pinf))
        bmax_n = jnp.where(m, mx, bmax_n)
        bmin_n = jnp.where(m, mn, bmin_n)
    probs = (hf - bmin_n) / (bmax_n + 1e-6 - bmin_n)
    p_ref[...] = probs
    x2_ref[...] = (probs - u > 0).astype(jnp.float32)


def _tc1(hf2, batch2, u2, degb):
    return pl.pallas_call(
        _tc1_body,
        out_shape=(jax.ShapeDtypeStruct((392, 128), jnp.float32),
                   jax.ShapeDtypeStruct((392, 128), jnp.float32),
                   jax.ShapeDtypeStruct((392, 128), jnp.float32)),
    )(hf2, batch2, u2, degb)


def _tc2_body(nb_ref, eb_ref, sb_ref, out_ref):
    nb = jnp.sum(nb_ref[...], axis=0)
    esum = jnp.sum(eb_ref[...], axis=0, keepdims=True)
    ssum = jnp.sum(sb_ref[...], axis=0, keepdims=True)
    lane = jax.lax.broadcasted_iota(jnp.int32, (1, 64), 1)
    valid = lane < _G
    card_1 = nb[0:1, :]
    self_sums = nb[1:2, :]
    set_size = nb[2:3, :]
    totalvol = nb[3:4, :] + 1e-6
    vol_hard = nb[4:5, :] + 1e-6
    graph_sums = card_1
    pairwise_prodsums = graph_sums * graph_sums / 2.0
    ewg = esum / 2.0
    sw = ssum / 2.0 + 1e-6
    ecw = pairwise_prodsums - self_sums
    edist = ecw - ewg
    ceh = set_size * (set_size - 1.0) / 2.0 + 1e-6
    cdh = sw / ceh
    tvr = vol_hard / totalvol
    loss = 0.25 * edist * 0.5 - 0.5 * ewg

    def vmean(v):
        return jnp.sum(jnp.where(valid, v, 0.0)) / jnp.float32(_G)

    ewg_mean = vmean(ewg)
    edist_mean = vmean(edist)
    tvr_mean = vmean(tvr)
    loss_mean = vmean(loss)
    scal = jnp.where(lane == 0, ewg_mean, 0.0)
    scal = jnp.where(lane == 1, edist_mean, scal)
    scal = jnp.where(lane == 2, tvr_mean, scal)
    scal = jnp.where(lane == 3, loss_mean, scal)
    out_ref[...] = jnp.concatenate(
        [card_1, loss, set_size, cdh, scal,
         ewg, edist, tvr], axis=0)


def _tc2(node_bins, ewg_bins, sw_bins):
    return pl.pallas_call(
        _tc2_body,
        out_shape=jax.ShapeDtypeStruct((8, 64), jnp.float32),
    )(node_bins, ewg_bins, sw_bins)


def _bn(h, gamma, beta):
    mu = h.mean(axis=0)
    var = h.var(axis=0)
    return (h - mu) / jnp.sqrt(var + 1e-5) * gamma + beta


def _gin_conv(h, row, col, p, n):
    agg = jax.ops.segment_sum(h[row], col, num_segments=n)
    z = (1.0 + p["eps"]) * h + agg
    z = jax.nn.relu(z @ p["W1"] + p["b1"])
    z = jax.nn.relu(z @ p["W2"] + p["b2"])
    return _bn(z, p["gamma"], p["beta"])


def _get_mask(m, row, col, n):
    prop = jax.ops.segment_max(m[row], col, num_segments=n)
    prop = jnp.where(jnp.isfinite(prop), prop, 0.0)
    return (prop > 0).astype(jnp.float32)


def _gnorm(h, batch, num_graphs):
    cnt = jax.ops.segment_sum(jnp.ones((h.shape[0],), jnp.float32), batch, num_segments=num_graphs)
    inv = 1.0 / jnp.sqrt(jnp.maximum(cnt, 1.0))
    return h * inv[batch][:, None]


def kernel(x, edge_index, batch, params):
    key = jax.random.key(42)
    row, col = edge_index[0], edge_index[1]
    n = x.shape[0]
    epad = _EPAD - row.shape[0]
    npad = _NPAD - n
    rowp = jnp.concatenate([row, jnp.full((epad,), _PADNODE, jnp.int32)])
    colp = jnp.concatenate([col, jnp.full((epad,), _PADNODE, jnp.int32)])
    xp = jnp.concatenate([x, jnp.zeros((npad,), jnp.float32)])
    masks_flat = _sc_masks(xp, rowp, colp)
    batchp = jnp.concatenate([batch, jnp.full((npad,), _PADG, jnp.int32)])
    cnt_bins = _sc_cnt(batchp)
    cnt = jnp.sum(cnt_bins, axis=0)
    inv = 1.0 / jnp.sqrt(jnp.maximum(cnt, 1.0))
    pernode2 = _tcinv(batchp.reshape(392, 128), inv.reshape(1, 64))
    pernode = pernode2.reshape(_NPAD)[:n][:, None]

    def mk(k):
        return (masks_flat[k * _NPAD:k * _NPAD + n] > 0).astype(jnp.float32)[:, None]

    xx = x[:, None]
    mask = mk(0)
    h = jax.nn.leaky_relu(_gin_conv(xx, row, col, params["conv1"], n))
    h = h * mask
    h = h * pernode
    h = _bn(h, params["bn1"]["gamma"], params["bn1"]["beta"])
    for li, (cp, bp) in enumerate(zip(params["convs"], params["bns"])):
        h = h + jax.nn.leaky_relu(_gin_conv(h, row, col, cp, n))
        mask = mk(li + 1)
        h = h * mask
        h = h * pernode
        h = _bn(h, bp["gamma"], bp["beta"])
    h = jax.nn.leaky_relu(h @ params["lin1"]["W"] + params["lin1"]["b"])
    h = h * mask
    h = jax.nn.leaky_relu(h @ params["lin2"]["W"] + params["lin2"]["b"])
    hf = h * mask

    hfp = jnp.concatenate([hf[:, 0], jnp.zeros((npad,), jnp.float32)])
    u = jax.random.uniform(key, (n, 1), jnp.float32)
    up = jnp.concatenate([u[:, 0], jnp.zeros((npad,), jnp.float32)])

    deg_bins = _sc_deg(rowp)
    probs2, x22, deg2 = _tc1(hfp.reshape(392, 128),
                             batchp.reshape(392, 128),
                             up.reshape(392, 128),
                             deg_bins.reshape(_NW, 392, 128))
    probs_f = probs2.reshape(_NPAD)
    x2_f = x22.reshape(_NPAD)
    deg_f = deg2.reshape(_NPAD)
    ewg_bins = _sc_edge(probs_f, batchp, rowp, colp)
    sw_bins = _sc_edge(x2_f, batchp, rowp, colp)
    node_bins = _sc_nodes(batchp, probs_f, x2_f, deg_f)
    out8 = _tc2(node_bins.reshape(_NW, 5, 64), ewg_bins, sw_bins)

    probs_out = probs_f[:n]
    card_1 = out8[0, :_G]
    loss = out8[1, :_G]
    set_size = out8[2, :_G]
    cdh = out8[3, :_G]
    ewg_mean = out8[4, 0]
    edist_mean = out8[4, 1]
    tvr_mean = out8[4, 2]
    loss_mean = out8[4, 3]
    return (probs_out, card_1, loss, set_size, ewg_mean, edist_mean,
            cdh, tvr_mean, loss_mean)


# speedup vs baseline: 3.7075x; 1.1291x over previous
import functools

import jax
import jax.numpy as jnp
from jax import lax
from jax.experimental import pallas as pl
from jax.experimental.pallas import tpu as pltpu
from jax.experimental.pallas import tpu_sc as plsc

_G = 50
_NC, _NS = 2, 16
_NW = _NC * _NS
_NPAD = 50176
_NPW = _NPAD // _NW
_EPAD = 800256
_EPW = _EPAD // _NW
_ECHUNK = 8336
_PADNODE = 50047
_PADG = _G

_mesh = plsc.VectorSubcoreMesh(core_axis_name="c", subcore_axis_name="s")


def _wid():
    return lax.axis_index("s") * _NC + lax.axis_index("c")


def _zero(ref, n):
    def body(i, c):
        ref[pl.ds(i * 16, 16)] = jnp.zeros((16,), ref.dtype)
        return c
    lax.fori_loop(0, n // 16, body, 0)


@functools.partial(
    pl.kernel,
    out_type=jax.ShapeDtypeStruct((_NW, 64), jnp.float32),
    mesh=_mesh,
    compiler_params=pltpu.CompilerParams(needs_layout_passes=False),
    scratch_types=[pltpu.VMEM((_NPAD,), jnp.float32),
                   pltpu.VMEM((_NPAD,), jnp.int32),
                   pltpu.VMEM((_ECHUNK,), jnp.int32),
                   pltpu.VMEM((_ECHUNK,), jnp.int32),
                   pltpu.VMEM((64,), jnp.float32)],
)
def _sc_edge(vals_hbm, batch_hbm, row_hbm, col_hbm, out_hbm,
             vals_v, batch_v, row_v, col_v, acc_v):
    w = _wid()
    pltpu.sync_copy(vals_hbm, vals_v)
    pltpu.sync_copy(batch_hbm, batch_v)
    _zero(acc_v, 64)
    base = w * _EPW

    def chunk(k, c):
        pltpu.sync_copy(row_hbm.at[pl.ds(base + k * _ECHUNK, _ECHUNK)], row_v)
        pltpu.sync_copy(col_hbm.at[pl.ds(base + k * _ECHUNK, _ECHUNK)], col_v)

        def body(i, c2):
            sl = pl.ds(i * 16, 16)
            r = row_v[sl]
            cc = col_v[sl]
            pr = plsc.load_gather(vals_v, [r])
            pc = plsc.load_gather(vals_v, [cc])
            b = plsc.load_gather(batch_v, [r])
            contrib = jnp.where(r != cc, pr * pc, jnp.zeros((16,), jnp.float32))
            plsc.addupdate_scatter(acc_v, [b], contrib)
            return c2
        lax.fori_loop(0, _ECHUNK // 16, body, c, unroll=8)
        return c
    lax.fori_loop(0, _EPW // _ECHUNK, chunk, 0)
    pltpu.sync_copy(acc_v, out_hbm.at[w])


@functools.partial(
    pl.kernel,
    out_type=jax.ShapeDtypeStruct((_NW, 320), jnp.float32),
    mesh=_mesh,
    compiler_params=pltpu.CompilerParams(needs_layout_passes=False),
    scratch_types=[pltpu.VMEM((_NPW,), jnp.int32),
                   pltpu.VMEM((_NPW,), jnp.float32),
                   pltpu.VMEM((_NPW,), jnp.float32),
                   pltpu.VMEM((_NPW,), jnp.float32),
                   pltpu.VMEM((320,), jnp.float32)],
)
def _sc_nodes(batch_hbm, p_hbm, x2_hbm, deg_hbm, out_hbm,
              b_v, p_v, x2_v, deg_v, acc_v):
    w = _wid()
    base = w * _NPW
    pltpu.sync_copy(batch_hbm.at[pl.ds(base, _NPW)], b_v)
    pltpu.sync_copy(p_hbm.at[pl.ds(base, _NPW)], p_v)
    pltpu.sync_copy(x2_hbm.at[pl.ds(base, _NPW)], x2_v)
    pltpu.sync_copy(deg_hbm.at[pl.ds(base, _NPW)], deg_v)
    _zero(acc_v, 320)

    def body(i, c):
        sl = pl.ds(i * 16, 16)
        b = b_v[sl]
        p = p_v[sl]
        x = x2_v[sl]
        d = deg_v[sl]
        plsc.addupdate_scatter(acc_v, [b], p)
        plsc.addupdate_scatter(acc_v, [b + 64], p * p)
        plsc.addupdate_scatter(acc_v, [b + 128], x)
        plsc.addupdate_scatter(acc_v, [b + 192], d)
        plsc.addupdate_scatter(acc_v, [b + 256], d * x)
        return c
    lax.fori_loop(0, _NPW // 16, body, 0, unroll=4)
    pltpu.sync_copy(acc_v, out_hbm.at[w])


_MNPT = _NPAD // _NS
_EPT = _EPAD // _NS

@functools.partial(
    pl.kernel,
    out_type=(jax.ShapeDtypeStruct((5 * _NPAD,), jnp.float32),
              jax.ShapeDtypeStruct((_NS * _NPAD,), jnp.float32)),
    mesh=_mesh,
    compiler_params=pltpu.CompilerParams(needs_layout_passes=False),
    scratch_types=[pltpu.VMEM((_NPAD,), jnp.float32),
                   pltpu.VMEM((_NPAD,), jnp.float32),
                   pltpu.VMEM((_ECHUNK,), jnp.int32),
                   pltpu.VMEM((_ECHUNK,), jnp.int32),
                   pltpu.VMEM((_MNPT,), jnp.float32),
                   pltpu.VMEM((_MNPT,), jnp.float32)],
)
def _sc_masks(x_hbm, row_hbm, col_hbm, out_hbm, parts_hbm,
              tab_v, acc_v, row_v, col_v, racc_v, tmp_v):
    core = lax.axis_index("c")
    tid = lax.axis_index("s")
    lo = tid * _MNPT
    ebase = tid * _EPT
    ones = jnp.ones((16,), jnp.float32)
    for k in range(5):
        @pl.when(core == 0)
        def _accumulate():
            if k == 0:
                pltpu.sync_copy(x_hbm, tab_v)
            elif k < 4:
                pltpu.sync_copy(out_hbm.at[pl.ds((k - 1) * _NPAD, _NPAD)], tab_v)
            _zero(acc_v, _NPAD)

            def chunk(j, c):
                off = ebase + j * _ECHUNK
                pltpu.sync_copy(row_hbm.at[pl.ds(off, _ECHUNK)], row_v)
                if k < 4:
                    pltpu.sync_copy(col_hbm.at[pl.ds(off, _ECHUNK)], col_v)

                def body(i, c2):
                    sl = pl.ds(i * 16, 16)
                    r = row_v[sl]
                    if k < 4:
                        cc = col_v[sl]
                        v = plsc.load_gather(tab_v, [r])
                        ind = (v > 0).astype(jnp.float32)
                        plsc.addupdate_scatter(acc_v, [cc], ind)
                    else:
                        plsc.addupdate_scatter(acc_v, [r], ones)
                    return c2
                lax.fori_loop(0, _ECHUNK // 16, body, c, unroll=8)
                return c
            lax.fori_loop(0, _EPT // _ECHUNK, chunk, 0)
            pltpu.sync_copy(acc_v, parts_hbm.at[pl.ds(tid * _NPAD, _NPAD)])
        plsc.subcore_barrier()

        @pl.when(core == 0)
        def _combine():
            _zero(racc_v, _MNPT)

            def comb(kk, c):
                pltpu.sync_copy(parts_hbm.at[pl.ds(kk * _NPAD + lo, _MNPT)], tmp_v)

                def addv(i, c2):
                    sl = pl.ds(i * 16, 16)
                    racc_v[sl] = racc_v[sl] + tmp_v[sl]
                    return c2
                lax.fori_loop(0, _MNPT // 16, addv, c, unroll=8)
                return c
            lax.fori_loop(0, _NS, comb, 0)
            pltpu.sync_copy(racc_v, out_hbm.at[pl.ds(k * _NPAD + lo, _MNPT)])
        plsc.subcore_barrier()


@functools.partial(
    pl.kernel,
    out_type=jax.ShapeDtypeStruct((_NW, 64), jnp.float32),
    mesh=_mesh,
    compiler_params=pltpu.CompilerParams(needs_layout_passes=False),
    scratch_types=[pltpu.VMEM((_NPW,), jnp.int32),
                   pltpu.VMEM((64,), jnp.float32)],
)
def _sc_cnt(batch_hbm, out_hbm, b_v, acc_v):
    w = _wid()
    pltpu.sync_copy(batch_hbm.at[pl.ds(w * _NPW, _NPW)], b_v)
    _zero(acc_v, 64)
    ones = jnp.ones((16,), jnp.float32)

    def body(i, c):
        b = b_v[pl.ds(i * 16, 16)]
        plsc.addupdate_scatter(acc_v, [b], ones)
        return c
    lax.fori_loop(0, _NPW // 16, body, 0, unroll=8)
    pltpu.sync_copy(acc_v, out_hbm.at[w])


def _tcinv_body(b_ref, inv_ref, o_ref):
    b = b_ref[...]
    inv = inv_ref[...]
    out = jnp.zeros((392, 128), jnp.float32)
    for g in range(_G):
        out = jnp.where(b == g, inv[0, g], out)
    o_ref[...] = out


def _tcinv(batch2, inv64):
    return pl.pallas_call(
        _tcinv_body,
        out_shape=jax.ShapeDtypeStruct((392, 128), jnp.float32),
    )(batch2, inv64)


def _tc1_body(hf_ref, b_ref, u_ref, p_ref, x2_ref):
    hf = hf_ref[...]
    b = b_ref[...]
    u = u_ref[...]
    bmax_n = jnp.zeros_like(hf)
    bmin_n = jnp.zeros_like(hf)
    ninf = jnp.float32(-jnp.inf)
    pinf = jnp.float32(jnp.inf)
    for g in range(_G):
        m = b == g
        mx = jnp.max(jnp.where(m, hf, ninf))
        mn = jnp.min(jnp.where(m, hf, pinf))
        bmax_n = jnp.where(m, mx, bmax_n)
        bmin_n = jnp.where(m, mn, bmin_n)
    probs = (hf - bmin_n) / (bmax_n + 1e-6 - bmin_n)
    p_ref[...] = probs
    x2_ref[...] = (probs - u > 0).astype(jnp.float32)


def _tc1(hf2, batch2, u2):
    return pl.pallas_call(
        _tc1_body,
        out_shape=(jax.ShapeDtypeStruct((392, 128), jnp.float32),
                   jax.ShapeDtypeStruct((392, 128), jnp.float32)),
    )(hf2, batch2, u2)


def _tc2_body(nb_ref, eb_ref, sb_ref, out_ref):
    nb = jnp.sum(nb_ref[...], axis=0)
    esum = jnp.sum(eb_ref[...], axis=0, keepdims=True)
    ssum = jnp.sum(sb_ref[...], axis=0, keepdims=True)
    lane = jax.lax.broadcasted_iota(jnp.int32, (1, 64), 1)
    valid = lane < _G
    card_1 = nb[0:1, :]
    self_sums = nb[1:2, :]
    set_size = nb[2:3, :]
    totalvol = nb[3:4, :] + 1e-6
    vol_hard = nb[4:5, :] + 1e-6
    graph_sums = card_1
    pairwise_prodsums = graph_sums * graph_sums / 2.0
    ewg = esum / 2.0
    sw = ssum / 2.0 + 1e-6
    ecw = pairwise_prodsums - self_sums
    edist = ecw - ewg
    ceh = set_size * (set_size - 1.0) / 2.0 + 1e-6
    cdh = sw / ceh
    tvr = vol_hard / totalvol
    loss = 0.25 * edist * 0.5 - 0.5 * ewg

    def vmean(v):
        return jnp.sum(jnp.where(valid, v, 0.0)) / jnp.float32(_G)

    ewg_mean = vmean(ewg)
    edist_mean = vmean(edist)
    tvr_mean = vmean(tvr)
    loss_mean = vmean(loss)
    scal = jnp.where(lane == 0, ewg_mean, 0.0)
    scal = jnp.where(lane == 1, edist_mean, scal)
    scal = jnp.where(lane == 2, tvr_mean, scal)
    scal = jnp.where(lane == 3, loss_mean, scal)
    out_ref[...] = jnp.concatenate(
        [card_1, loss, set_size, cdh, scal,
         ewg, edist, tvr], axis=0)


def _tc2(node_bins, ewg_bins, sw_bins):
    return pl.pallas_call(
        _tc2_body,
        out_shape=jax.ShapeDtypeStruct((8, 64), jnp.float32),
    )(node_bins, ewg_bins, sw_bins)


def _bn(h, gamma, beta):
    mu = h.mean(axis=0)
    var = h.var(axis=0)
    return (h - mu) / jnp.sqrt(var + 1e-5) * gamma + beta


def _gin_conv(h, row, col, p, n):
    agg = jax.ops.segment_sum(h[row], col, num_segments=n)
    z = (1.0 + p["eps"]) * h + agg
    z = jax.nn.relu(z @ p["W1"] + p["b1"])
    z = jax.nn.relu(z @ p["W2"] + p["b2"])
    return _bn(z, p["gamma"], p["beta"])


def _get_mask(m, row, col, n):
    prop = jax.ops.segment_max(m[row], col, num_segments=n)
    prop = jnp.where(jnp.isfinite(prop), prop, 0.0)
    return (prop > 0).astype(jnp.float32)


def _gnorm(h, batch, num_graphs):
    cnt = jax.ops.segment_sum(jnp.ones((h.shape[0],), jnp.float32), batch, num_segments=num_graphs)
    inv = 1.0 / jnp.sqrt(jnp.maximum(cnt, 1.0))
    return h * inv[batch][:, None]


def kernel(x, edge_index, batch, params):
    key = jax.random.key(42)
    row, col = edge_index[0], edge_index[1]
    n = x.shape[0]
    epad = _EPAD - row.shape[0]
    npad = _NPAD - n
    rowp = jnp.concatenate([row, jnp.full((epad,), _PADNODE, jnp.int32)])
    colp = jnp.concatenate([col, jnp.full((epad,), _PADNODE, jnp.int32)])
    xp = jnp.concatenate([x, jnp.zeros((npad,), jnp.float32)])
    masks_flat, _ = _sc_masks(xp, rowp, colp)
    batchp = jnp.concatenate([batch, jnp.full((npad,), _PADG, jnp.int32)])
    cnt_bins = _sc_cnt(batchp)
    cnt = jnp.sum(cnt_bins, axis=0)
    inv = 1.0 / jnp.sqrt(jnp.maximum(cnt, 1.0))
    pernode2 = _tcinv(batchp.reshape(392, 128), inv.reshape(1, 64))
    pernode = pernode2.reshape(_NPAD)[:n][:, None]

    def mk(k):
        return (masks_flat[k * _NPAD:k * _NPAD + n] > 0).astype(jnp.float32)[:, None]

    xx = x[:, None]
    mask = mk(0)
    h = jax.nn.leaky_relu(_gin_conv(xx, row, col, params["conv1"], n))
    h = h * mask
    h = h * pernode
    h = _bn(h, params["bn1"]["gamma"], params["bn1"]["beta"])
    for li, (cp, bp) in enumerate(zip(params["convs"], params["bns"])):
        h = h + jax.nn.leaky_relu(_gin_conv(h, row, col, cp, n))
        mask = mk(li + 1)
        h = h * mask
        h = h * pernode
        h = _bn(h, bp["gamma"], bp["beta"])
    h = jax.nn.leaky_relu(h @ params["lin1"]["W"] + params["lin1"]["b"])
    h = h * mask
    h = jax.nn.leaky_relu(h @ params["lin2"]["W"] + params["lin2"]["b"])
    hf = h * mask

    hfp = jnp.concatenate([hf[:, 0], jnp.zeros((npad,), jnp.float32)])
    u = jax.random.uniform(key, (n, 1), jnp.float32)
    up = jnp.concatenate([u[:, 0], jnp.zeros((npad,), jnp.float32)])

    probs2, x22 = _tc1(hfp.reshape(392, 128),
                       batchp.reshape(392, 128),
                       up.reshape(392, 128))
    probs_f = probs2.reshape(_NPAD)
    x2_f = x22.reshape(_NPAD)
    deg_f = masks_flat[4 * _NPAD:]
    ewg_bins = _sc_edge(probs_f, batchp, rowp, colp)
    sw_bins = _sc_edge(x2_f, batchp, rowp, colp)
    node_bins = _sc_nodes(batchp, probs_f, x2_f, deg_f)
    out8 = _tc2(node_bins.reshape(_NW, 5, 64), ewg_bins, sw_bins)

    probs_out = probs_f[:n]
    card_1 = out8[0, :_G]
    loss = out8[1, :_G]
    set_size = out8[2, :_G]
    cdh = out8[3, :_G]
    ewg_mean = out8[4, 0]
    edist_mean = out8[4, 1]
    tvr_mean = out8[4, 2]
    loss_mean = out8[4, 3]
    return (probs_out, card_1, loss, set_size, ewg_mean, edist_mean,
            cdh, tvr_mean, loss_mean)


# speedup vs baseline: 3.7135x; 1.0016x over previous
import functools

import jax
import jax.numpy as jnp
from jax import lax
from jax.experimental import pallas as pl
from jax.experimental.pallas import tpu as pltpu
from jax.experimental.pallas import tpu_sc as plsc

_G = 50
_NC, _NS = 2, 16
_NW = _NC * _NS
_NPAD = 50176
_NPW = _NPAD // _NW
_EPAD = 800256
_EPW = _EPAD // _NW
_ECHUNK = 8336
_PADNODE = 50047
_PADG = _G

_mesh = plsc.VectorSubcoreMesh(core_axis_name="c", subcore_axis_name="s")


def _wid():
    return lax.axis_index("s") * _NC + lax.axis_index("c")


def _zero(ref, n):
    def body(i, c):
        ref[pl.ds(i * 16, 16)] = jnp.zeros((16,), ref.dtype)
        return c
    lax.fori_loop(0, n // 16, body, 0)


@functools.partial(
    pl.kernel,
    out_type=jax.ShapeDtypeStruct((_NW, 448), jnp.float32),
    mesh=_mesh,
    compiler_params=pltpu.CompilerParams(needs_layout_passes=False),
    scratch_types=[pltpu.VMEM((_NPAD,), jnp.float32),
                   pltpu.VMEM((_NPAD,), jnp.int32),
                   pltpu.VMEM((_ECHUNK,), jnp.int32),
                   pltpu.VMEM((_ECHUNK,), jnp.int32),
                   pltpu.VMEM((_NPW,), jnp.float32),
                   pltpu.VMEM((448,), jnp.float32)],
)
def _sc_post(comb_hbm, batch_hbm, row_hbm, col_hbm, deg_hbm, out_hbm,
             tab_v, batch_v, row_v, col_v, deg_v, acc_v):
    w = _wid()
    pltpu.sync_copy(comb_hbm, tab_v)
    pltpu.sync_copy(batch_hbm, batch_v)
    pltpu.sync_copy(deg_hbm.at[pl.ds(w * _NPW, _NPW)], deg_v)
    _zero(acc_v, 448)
    ebase = w * _EPW

    def chunk(k, c):
        pltpu.sync_copy(row_hbm.at[pl.ds(ebase + k * _ECHUNK, _ECHUNK)], row_v)
        pltpu.sync_copy(col_hbm.at[pl.ds(ebase + k * _ECHUNK, _ECHUNK)], col_v)

        def body(i, c2):
            sl = pl.ds(i * 16, 16)
            r = row_v[sl]
            cc = col_v[sl]
            gr = plsc.load_gather(tab_v, [r])
            gc = plsc.load_gather(tab_v, [cc])
            b = plsc.load_gather(batch_v, [r])
            pr = jnp.abs(gr)
            pc = jnp.abs(gc)
            xr = lax.shift_right_logical(plsc.bitcast(gr, jnp.int32), jnp.int32(31))
            xc = lax.shift_right_logical(plsc.bitcast(gc, jnp.int32), jnp.int32(31))
            xprod = (xr * xc).astype(jnp.float32)
            nz = r != cc
            zero = jnp.zeros((16,), jnp.float32)
            plsc.addupdate_scatter(acc_v, [b], jnp.where(nz, pr * pc, zero))
            plsc.addupdate_scatter(acc_v, [b + 64], jnp.where(nz, xprod, zero))
            return c2
        lax.fori_loop(0, _ECHUNK // 16, body, c, unroll=8)
        return c
    lax.fori_loop(0, _EPW // _ECHUNK, chunk, 0)

    nbase = w * _NPW

    def nodes(i, c):
        sl = pl.ds(i * 16, 16)
        g = tab_v[pl.ds(nbase + i * 16, 16)]
        b = batch_v[pl.ds(nbase + i * 16, 16)]
        d = deg_v[sl]
        p = jnp.abs(g)
        x = lax.shift_right_logical(plsc.bitcast(g, jnp.int32), jnp.int32(31)).astype(jnp.float32)
        plsc.addupdate_scatter(acc_v, [b + 128], p)
        plsc.addupdate_scatter(acc_v, [b + 192], p * p)
        plsc.addupdate_scatter(acc_v, [b + 256], x)
        plsc.addupdate_scatter(acc_v, [b + 320], d)
        plsc.addupdate_scatter(acc_v, [b + 384], d * x)
        return c
    lax.fori_loop(0, _NPW // 16, nodes, 0, unroll=4)
    pltpu.sync_copy(acc_v, out_hbm.at[w])


_MNPT = _NPAD // _NS
_EPT = _EPAD // _NS

@functools.partial(
    pl.kernel,
    out_type=(jax.ShapeDtypeStruct((5 * _NPAD,), jnp.float32),
              jax.ShapeDtypeStruct((_NS * _NPAD,), jnp.float32)),
    mesh=_mesh,
    compiler_params=pltpu.CompilerParams(needs_layout_passes=False),
    scratch_types=[pltpu.VMEM((_NPAD,), jnp.float32),
                   pltpu.VMEM((_NPAD,), jnp.float32),
                   pltpu.VMEM((_ECHUNK,), jnp.int32),
                   pltpu.VMEM((_ECHUNK,), jnp.int32),
                   pltpu.VMEM((_MNPT,), jnp.float32),
                   pltpu.VMEM((_MNPT,), jnp.float32)],
)
def _sc_masks(x_hbm, row_hbm, col_hbm, out_hbm, parts_hbm,
              tab_v, acc_v, row_v, col_v, racc_v, tmp_v):
    core = lax.axis_index("c")
    tid = lax.axis_index("s")
    lo = tid * _MNPT
    ebase = tid * _EPT
    ones = jnp.ones((16,), jnp.float32)
    for k in range(5):
        @pl.when(core == 0)
        def _accumulate():
            if k == 0:
                pltpu.sync_copy(x_hbm, tab_v)
            elif k < 4:
                pltpu.sync_copy(out_hbm.at[pl.ds((k - 1) * _NPAD, _NPAD)], tab_v)
            _zero(acc_v, _NPAD)

            def chunk(j, c):
                off = ebase + j * _ECHUNK
                pltpu.sync_copy(row_hbm.at[pl.ds(off, _ECHUNK)], row_v)
                if k < 4:
                    pltpu.sync_copy(col_hbm.at[pl.ds(off, _ECHUNK)], col_v)

                def body(i, c2):
                    sl = pl.ds(i * 16, 16)
                    r = row_v[sl]
                    if k < 4:
                        cc = col_v[sl]
                        v = plsc.load_gather(tab_v, [r])
                        ind = (v > 0).astype(jnp.float32)
                        plsc.addupdate_scatter(acc_v, [cc], ind)
                    else:
                        plsc.addupdate_scatter(acc_v, [r], ones)
                    return c2
                lax.fori_loop(0, _ECHUNK // 16, body, c, unroll=8)
                return c
            lax.fori_loop(0, _EPT // _ECHUNK, chunk, 0)
            pltpu.sync_copy(acc_v, parts_hbm.at[pl.ds(tid * _NPAD, _NPAD)])
        plsc.subcore_barrier()

        @pl.when(core == 0)
        def _combine():
            _zero(racc_v, _MNPT)

            def comb(kk, c):
                pltpu.sync_copy(parts_hbm.at[pl.ds(kk * _NPAD + lo, _MNPT)], tmp_v)

                def addv(i, c2):
                    sl = pl.ds(i * 16, 16)
                    racc_v[sl] = racc_v[sl] + tmp_v[sl]
                    return c2
                lax.fori_loop(0, _MNPT // 16, addv, c, unroll=8)
                return c
            lax.fori_loop(0, _NS, comb, 0)
            pltpu.sync_copy(racc_v, out_hbm.at[pl.ds(k * _NPAD + lo, _MNPT)])
        plsc.subcore_barrier()


@functools.partial(
    pl.kernel,
    out_type=jax.ShapeDtypeStruct((_NW, 64), jnp.float32),
    mesh=_mesh,
    compiler_params=pltpu.CompilerParams(needs_layout_passes=False),
    scratch_types=[pltpu.VMEM((_NPW,), jnp.int32),
                   pltpu.VMEM((64,), jnp.float32)],
)
def _sc_cnt(batch_hbm, out_hbm, b_v, acc_v):
    w = _wid()
    pltpu.sync_copy(batch_hbm.at[pl.ds(w * _NPW, _NPW)], b_v)
    _zero(acc_v, 64)
    ones = jnp.ones((16,), jnp.float32)

    def body(i, c):
        b = b_v[pl.ds(i * 16, 16)]
        plsc.addupdate_scatter(acc_v, [b], ones)
        return c
    lax.fori_loop(0, _NPW // 16, body, 0, unroll=8)
    pltpu.sync_copy(acc_v, out_hbm.at[w])


def _tcinv_body(b_ref, inv_ref, o_ref):
    b = b_ref[...]
    inv = inv_ref[...]
    out = jnp.zeros((392, 128), jnp.float32)
    for g in range(_G):
        out = jnp.where(b == g, inv[0, g], out)
    o_ref[...] = out


def _tcinv(batch2, inv64):
    return pl.pallas_call(
        _tcinv_body,
        out_shape=jax.ShapeDtypeStruct((392, 128), jnp.float32),
    )(batch2, inv64)


def _tc1_body(hf_ref, b_ref, u_ref, p_ref, comb_ref):
    hf = hf_ref[...]
    b = b_ref[...]
    u = u_ref[...]
    bmax_n = jnp.zeros_like(hf)
    bmin_n = jnp.zeros_like(hf)
    ninf = jnp.float32(-jnp.inf)
    pinf = jnp.float32(jnp.inf)
    for g in range(_G):
        m = b == g
        mx = jnp.max(jnp.where(m, hf, ninf))
        mn = jnp.min(jnp.where(m, hf, pinf))
        bmax_n = jnp.where(m, mx, bmax_n)
        bmin_n = jnp.where(m, mn, bmin_n)
    probs = (hf - bmin_n) / (bmax_n + 1e-6 - bmin_n)
    p_ref[...] = probs
    x2b = probs - u > 0
    comb_ref[...] = jnp.where(x2b, -probs, probs)


def _tc1(hf2, batch2, u2):
    return pl.pallas_call(
        _tc1_body,
        out_shape=(jax.ShapeDtypeStruct((392, 128), jnp.float32),
                   jax.ShapeDtypeStruct((392, 128), jnp.float32)),
    )(hf2, batch2, u2)


def _tc2_body(nb_ref, out_ref):
    nb = jnp.sum(nb_ref[...], axis=0)
    esum = nb[0:1, :]
    ssum = nb[1:2, :]
    lane = jax.lax.broadcasted_iota(jnp.int32, (1, 64), 1)
    valid = lane < _G
    card_1 = nb[2:3, :]
    self_sums = nb[3:4, :]
    set_size = nb[4:5, :]
    totalvol = nb[5:6, :] + 1e-6
    vol_hard = nb[6:7, :] + 1e-6
    graph_sums = card_1
    pairwise_prodsums = graph_sums * graph_sums / 2.0
    ewg = esum / 2.0
    sw = ssum / 2.0 + 1e-6
    ecw = pairwise_prodsums - self_sums
    edist = ecw - ewg
    ceh = set_size * (set_size - 1.0) / 2.0 + 1e-6
    cdh = sw / ceh
    tvr = vol_hard / totalvol
    loss = 0.25 * edist * 0.5 - 0.5 * ewg

    def vmean(v):
        return jnp.sum(jnp.where(valid, v, 0.0)) / jnp.float32(_G)

    ewg_mean = vmean(ewg)
    edist_mean = vmean(edist)
    tvr_mean = vmean(tvr)
    loss_mean = vmean(loss)
    scal = jnp.where(lane == 0, ewg_mean, 0.0)
    scal = jnp.where(lane == 1, edist_mean, scal)
    scal = jnp.where(lane == 2, tvr_mean, scal)
    scal = jnp.where(lane == 3, loss_mean, scal)
    out_ref[...] = jnp.concatenate(
        [card_1, loss, set_size, cdh, scal,
         ewg, edist, tvr], axis=0)


def _tc2(post_bins):
    return pl.pallas_call(
        _tc2_body,
        out_shape=jax.ShapeDtypeStruct((8, 64), jnp.float32),
    )(post_bins)


def _bn(h, gamma, beta):
    mu = h.mean(axis=0)
    var = h.var(axis=0)
    return (h - mu) / jnp.sqrt(var + 1e-5) * gamma + beta


def _gin_conv(h, row, col, p, n):
    agg = jax.ops.segment_sum(h[row], col, num_segments=n)
    z = (1.0 + p["eps"]) * h + agg
    z = jax.nn.relu(z @ p["W1"] + p["b1"])
    z = jax.nn.relu(z @ p["W2"] + p["b2"])
    return _bn(z, p["gamma"], p["beta"])


def _get_mask(m, row, col, n):
    prop = jax.ops.segment_max(m[row], col, num_segments=n)
    prop = jnp.where(jnp.isfinite(prop), prop, 0.0)
    return (prop > 0).astype(jnp.float32)


def _gnorm(h, batch, num_graphs):
    cnt = jax.ops.segment_sum(jnp.ones((h.shape[0],), jnp.float32), batch, num_segments=num_graphs)
    inv = 1.0 / jnp.sqrt(jnp.maximum(cnt, 1.0))
    return h * inv[batch][:, None]


def kernel(x, edge_index, batch, params):
    key = jax.random.key(42)
    row, col = edge_index[0], edge_index[1]
    n = x.shape[0]
    epad = _EPAD - row.shape[0]
    npad = _NPAD - n
    rowp = jnp.concatenate([row, jnp.full((epad,), _PADNODE, jnp.int32)])
    colp = jnp.concatenate([col, jnp.full((epad,), _PADNODE, jnp.int32)])
    xp = jnp.concatenate([x, jnp.zeros((npad,), jnp.float32)])
    masks_flat, _ = _sc_masks(xp, rowp, colp)
    batchp = jnp.concatenate([batch, jnp.full((npad,), _PADG, jnp.int32)])
    cnt_bins = _sc_cnt(batchp)
    cnt = jnp.sum(cnt_bins, axis=0)
    inv = 1.0 / jnp.sqrt(jnp.maximum(cnt, 1.0))
    pernode2 = _tcinv(batchp.reshape(392, 128), inv.reshape(1, 64))
    pernode = pernode2.reshape(_NPAD)[:n][:, None]

    def mk(k):
        return (masks_flat[k * _NPAD:k * _NPAD + n] > 0).astype(jnp.float32)[:, None]

    xx = x[:, None]
    mask = mk(0)
    h = jax.nn.leaky_relu(_gin_conv(xx, row, col, params["conv1"], n))
    h = h * mask
    h = h * pernode
    h = _bn(h, params["bn1"]["gamma"], params["bn1"]["beta"])
    for li, (cp, bp) in enumerate(zip(params["convs"], params["bns"])):
        h = h + jax.nn.leaky_relu(_gin_conv(h, row, col, cp, n))
        mask = mk(li + 1)
        h = h * mask
        h = h * pernode
        h = _bn(h, bp["gamma"], bp["beta"])
    h = jax.nn.leaky_relu(h @ params["lin1"]["W"] + params["lin1"]["b"])
    h = h * mask
    h = jax.nn.leaky_relu(h @ params["lin2"]["W"] + params["lin2"]["b"])
    hf = h * mask

    hfp = jnp.concatenate([hf[:, 0], jnp.zeros((npad,), jnp.float32)])
    u = jax.random.uniform(key, (n, 1), jnp.float32)
    up = jnp.concatenate([u[:, 0], jnp.zeros((npad,), jnp.float32)])

    probs2, comb2 = _tc1(hfp.reshape(392, 128),
                         batchp.reshape(392, 128),
                         up.reshape(392, 128))
    probs_f = probs2.reshape(_NPAD)
    comb_f = comb2.reshape(_NPAD)
    deg_f = masks_flat[4 * _NPAD:]
    post_bins = _sc_post(comb_f, batchp, rowp, colp, deg_f)
    out8 = _tc2(post_bins.reshape(_NW, 7, 64))

    probs_out = probs_f[:n]
    card_1 = out8[0, :_G]
    loss = out8[1, :_G]
    set_size = out8[2, :_G]
    cdh = out8[3, :_G]
    ewg_mean = out8[4, 0]
    edist_mean = out8[4, 1]
    tvr_mean = out8[4, 2]
    loss_mean = out8[4, 3]
    return (probs_out, card_1, loss, set_size, ewg_mean, edist_mean,
            cdh, tvr_mean, loss_mean)
